# baseline (speedup 1.0000x reference)
import jax
import jax.numpy as jnp
from jax import lax
from jax.experimental import pallas as pl
from jax.experimental.pallas import tpu as pltpu
from jax.experimental.pallas import tpu_sc as plsc

_NW = 32
_CR = 16
_NBUF = 2


def _make_sc_kernel(R, D, S):
    rows_w = R // _NW
    nch = rows_w // _CR
    wpb = S // rows_w

    def body(x_hbm, t_hbm, o_hbm, xbuf, tbuf,
             xs0, xs1, ts0, ts1, os0, os1):
        c = lax.axis_index("c")
        s = lax.axis_index("s")
        wid = s * 2 + c
        base = wid * rows_w
        tbase = (wid % wpb) * rows_w

        xsems = (xs0, xs1)
        tsems = (ts0, ts1)
        osems = (os0, os1)

        def xcopy(i, b):
            return pltpu.make_async_copy(
                x_hbm.at[pl.ds(base + i * _CR, _CR)], xbuf.at[b], xsems[b])

        def tcopy(i, b):
            return pltpu.make_async_copy(
                t_hbm.at[pl.ds(tbase + i * _CR, _CR)], tbuf.at[b], tsems[b])

        def ocopy(i, b):
            return pltpu.make_async_copy(
                xbuf.at[b], o_hbm.at[pl.ds(base + i * _CR, _CR)], osems[b])

        for b in range(_NBUF):
            xcopy(b, b).start()
            tcopy(b, b).start()

        def group(g, carry):
            for b in range(_NBUF):
                i = g * _NBUF + b
                pl.when(i >= _NBUF)(lambda: ocopy(i - _NBUF, b).wait())
                xcopy(i, b).wait()
                tcopy(i, b).wait()

                def row(r, carry2):
                    for j in range(D // 16):
                        sl = pl.ds(j * 16, 16)
                        plsc.addupdate(xbuf.at[b, r, sl], tbuf[b, r, sl])
                    return carry2

                lax.fori_loop(0, _CR, row, 0, unroll=False)
                ocopy(i, b).start()
                def prefetch():
                    xcopy(i + _NBUF, b).start()
                    tcopy(i + _NBUF, b).start()

                pl.when(i + _NBUF < nch)(prefetch)
            return carry

        lax.fori_loop(0, nch // _NBUF, group, 0, unroll=False)
        for b in range(_NBUF):
            ocopy(nch - _NBUF + b, b).wait()

    return pl.kernel(
        body,
        out_type=jax.ShapeDtypeStruct((R, D), jnp.float32),
        mesh=plsc.VectorSubcoreMesh(core_axis_name="c", subcore_axis_name="s"),
        scratch_types=[
            pltpu.VMEM((_NBUF, _CR, D), jnp.float32),
            pltpu.VMEM((_NBUF, _CR, D), jnp.float32),
            pltpu.SemaphoreType.DMA,
            pltpu.SemaphoreType.DMA,
            pltpu.SemaphoreType.DMA,
            pltpu.SemaphoreType.DMA,
            pltpu.SemaphoreType.DMA,
            pltpu.SemaphoreType.DMA,
        ],
    )


def kernel(x, embed_table):
    B, S, D = x.shape
    x2 = x.reshape(B * S, D)
    out = _make_sc_kernel(B * S, D, S)(x2, embed_table)
    return out.reshape(B, S, D)


# speedup vs baseline: 1.5676x; 1.5676x over previous
import jax
import jax.numpy as jnp
from jax import lax
from jax.experimental import pallas as pl
from jax.experimental.pallas import tpu as pltpu
from jax.experimental.pallas import tpu_sc as plsc

_NW = 32
_CR = 8
_NBUF = 4


def _make_sc_kernel(R, D, S):
    rows_w = R // _NW
    nch = rows_w // _CR
    wpb = S // rows_w
    groups = nch // _NBUF

    def body(x_hbm, t_hbm, o_hbm, xbuf, tbuf, *sems):
        xsems = sems[:_NBUF]
        tsems = sems[_NBUF:2 * _NBUF]
        osems = sems[2 * _NBUF:]
        c = lax.axis_index("c")
        s = lax.axis_index("s")
        wid = s * 2 + c
        base = wid * rows_w
        tbase = (wid % wpb) * rows_w

        def xcopy(i, b):
            return pltpu.make_async_copy(
                x_hbm.at[pl.ds(base + i * _CR, _CR)], xbuf.at[b], xsems[b])

        def tcopy(i, b):
            return pltpu.make_async_copy(
                t_hbm.at[pl.ds(tbase + i * _CR, _CR)], tbuf.at[b], tsems[b])

        def ocopy(i, b):
            return pltpu.make_async_copy(
                xbuf.at[b], o_hbm.at[pl.ds(base + i * _CR, _CR)], osems[b])

        for b in range(_NBUF):
            xcopy(b, b).start()
            tcopy(b, b).start()

        def group(g, carry):
            for b in range(_NBUF):
                i = g * _NBUF + b
                pl.when(i >= _NBUF)(lambda: ocopy(i - _NBUF, b).wait())
                xcopy(i, b).wait()
                tcopy(i, b).wait()

                def row(r, carry2):
                    for j in range(D // 16):
                        sl = pl.ds(j * 16, 16)
                        plsc.addupdate(xbuf.at[b, r, sl], tbuf[b, r, sl])
                    return carry2

                lax.fori_loop(0, _CR, row, 0, unroll=False)
                ocopy(i, b).start()

                def prefetch():
                    xcopy(i + _NBUF, b).start()
                    tcopy(i + _NBUF, b).start()

                pl.when(i + _NBUF < nch)(prefetch)
            return carry

        lax.fori_loop(0, groups, group, 0, unroll=False)
        for b in range(_NBUF):
            ocopy(nch - _NBUF + b, b).wait()

    return pl.kernel(
        body,
        out_type=jax.ShapeDtypeStruct((R, D), jnp.float32),
        mesh=plsc.VectorSubcoreMesh(core_axis_name="c", subcore_axis_name="s"),
        scratch_types=[
            pltpu.VMEM((_NBUF, _CR, D), jnp.float32),
            pltpu.VMEM((_NBUF, _CR, D), jnp.float32),
        ] + [pltpu.SemaphoreType.DMA] * (3 * _NBUF),
    )


def kernel(x, embed_table):
    B, S, D = x.shape
    x2 = x.reshape(B * S, D)
    out = _make_sc_kernel(B * S, D, S)(x2, embed_table)
    return out.reshape(B, S, D)


# speedup vs baseline: 1.9212x; 1.2256x over previous
import jax
import jax.numpy as jnp
from jax import lax
from jax.experimental import pallas as pl
from jax.experimental.pallas import tpu as pltpu
from jax.experimental.pallas import tpu_sc as plsc

_NW = 32
_CR = 8
_NBUF = 2


def _make_sc_kernel(B, S, D):
    tr = S // _NW
    nch = tr // _CR
    groups = nch // _NBUF

    def body(x_hbm, t_hbm, o_hbm, xbuf, tbuf, *sems):
        xsems = sems[:_NBUF]
        tsems = sems[_NBUF:2 * _NBUF]
        osems = sems[2 * _NBUF:]
        c = lax.axis_index("c")
        s = lax.axis_index("s")
        wid = s * 2 + c
        tb0 = wid * tr

        def tcopy(i, b):
            return pltpu.make_async_copy(
                t_hbm.at[pl.ds(tb0 + i * _CR, _CR)], tbuf.at[b], tsems[b])

        def xcopy(i, b, bb):
            return pltpu.make_async_copy(
                x_hbm.at[pl.ds(bb * S + tb0 + i * _CR, _CR)],
                xbuf.at[b, bb], xsems[b])

        def ocopy(i, b, bb):
            return pltpu.make_async_copy(
                xbuf.at[b, bb],
                o_hbm.at[pl.ds(bb * S + tb0 + i * _CR, _CR)], osems[b])

        for b in range(_NBUF):
            tcopy(b, b).start()
            for bb in range(B):
                xcopy(b, b, bb).start()

        def group(g, carry):
            for b in range(_NBUF):
                i = g * _NBUF + b
                def drain():
                    for bb in range(B):
                        ocopy(i - _NBUF, b, bb).wait()
                pl.when(i >= _NBUF)(drain)
                tcopy(i, b).wait()
                for bb in range(B):
                    xcopy(i, b, bb).wait()

                    def row(r, carry2):
                        for j in range(D // 16):
                            sl = pl.ds(j * 16, 16)
                            plsc.addupdate(xbuf.at[b, bb, r, sl],
                                           tbuf[b, r, sl])
                        return carry2

                    lax.fori_loop(0, _CR, row, 0, unroll=False)
                    ocopy(i, b, bb).start()

                def prefetch():
                    tcopy(i + _NBUF, b).start()
                    for bb in range(B):
                        xcopy(i + _NBUF, b, bb).start()

                pl.when(i + _NBUF < nch)(prefetch)
            return carry

        lax.fori_loop(0, groups, group, 0, unroll=False)
        for b in range(_NBUF):
            for bb in range(B):
                ocopy(nch - _NBUF + b, b, bb).wait()

    return pl.kernel(
        body,
        out_type=jax.ShapeDtypeStruct((B * S, D), jnp.float32),
        mesh=plsc.VectorSubcoreMesh(core_axis_name="c", subcore_axis_name="s"),
        scratch_types=[
            pltpu.VMEM((_NBUF, B, _CR, D), jnp.float32),
            pltpu.VMEM((_NBUF, _CR, D), jnp.float32),
        ] + [pltpu.SemaphoreType.DMA] * (3 * _NBUF),
    )


def kernel(x, embed_table):
    B, S, D = x.shape
    x2 = x.reshape(B * S, D)
    out = _make_sc_kernel(B, S, D)(x2, embed_table)
    return out.reshape(B, S, D)
